# baseline (speedup 1.0000x reference)
import functools

import jax
import jax.numpy as jnp
from jax import lax
from jax.experimental import pallas as pl
from jax.experimental.pallas import tpu as pltpu
from jax.experimental.pallas import tpu_sc as plsc


def _sc_gather(table, flat_idx):
    info = plsc.get_sparse_core_info()
    NC, NS = info.num_cores, info.num_subcores
    NW = NC * NS
    N = flat_idx.shape[0]
    E = table.shape[1]
    b_per_w = N // NW
    CH = 80
    n_ch = b_per_w // CH
    assert b_per_w * NW == N and n_ch * CH == b_per_w

    mesh = plsc.VectorSubcoreMesh(core_axis_name="c", subcore_axis_name="s")

    @functools.partial(
        pl.kernel,
        out_type=jax.ShapeDtypeStruct((N, E), table.dtype),
        mesh=mesh,
        scratch_types=[
            pltpu.VMEM((b_per_w,), jnp.int32),
            pltpu.VMEM((2, CH, E), table.dtype),
            pltpu.SemaphoreType.DMA,
            pltpu.SemaphoreType.DMA,
        ],
        compiler_params=pltpu.CompilerParams(use_tc_tiling_on_sc=False),
    )
    def k(tab_hbm, idx_hbm, out_hbm, idx_v, rows_v, sem0, sem1):
        wid = lax.axis_index("s") * NC + lax.axis_index("c")
        base = wid * b_per_w
        pltpu.sync_copy(idx_hbm.at[pl.ds(base, b_per_w)], idx_v)
        sems = (sem0, sem1)

        def start(c, slot):
            pltpu.async_copy(
                tab_hbm.at[idx_v.at[pl.ds(c * CH, CH)]],
                rows_v.at[slot],
                sems[slot],
            )

        def finish(c, slot):
            pltpu.make_async_copy(
                tab_hbm.at[idx_v.at[pl.ds(c * CH, CH)]],
                rows_v.at[slot],
                sems[slot],
            ).wait()
            pltpu.sync_copy(
                rows_v.at[slot], out_hbm.at[pl.ds(base + c * CH, CH)]
            )

        start(0, 0)
        start(1, 1)

        def body(g, carry):
            for j in range(2):
                c = 2 * g + j
                finish(c, j)

                @pl.when(c + 2 < n_ch)
                def _():
                    start(c + 2, j)

            return carry

        lax.fori_loop(0, n_ch // 2, body, 0)

    return k(table, flat_idx)


def _proj_body(w_ref, g_ref, b_ref, out_ref):
    g = g_ref[0]
    acc = lax.dot_general(
        w_ref[...], g,
        dimension_numbers=(((1,), (1,)), ((), ())),
        preferred_element_type=jnp.float32,
    )
    out_ref[0] = acc + b_ref[...]


def _proj_body_alias(w_ref, g_ref, b_ref, alias_ref, out_ref):
    del alias_ref
    _proj_body(w_ref, g_ref, b_ref, out_ref)


def _tc_proj_chunk(Wb, Gk, b2, S, off, out_prev=None):
    S_CH, Bb, E = Gk.shape
    V = Wb.shape[0]
    in_specs = [
        pl.BlockSpec((V, E), lambda s: (0, 0)),
        pl.BlockSpec((1, Bb, E), lambda s: (s, 0, 0)),
        pl.BlockSpec((V, 1), lambda s: (0, 0)),
    ]
    args = [Wb, Gk, b2]
    body = _proj_body
    aliases = {}
    if out_prev is not None:
        in_specs.append(pl.BlockSpec(memory_space=pl.ANY))
        args.append(out_prev)
        body = _proj_body_alias
        aliases = {3: 0}
    return pl.pallas_call(
        body,
        grid=(S_CH,),
        in_specs=in_specs,
        out_specs=pl.BlockSpec((1, V, Bb), lambda s, off=off: (s + off, 0, 0)),
        out_shape=jax.ShapeDtypeStruct((S, V, Bb), jnp.float32),
        input_output_aliases=aliases,
    )(*args)


def kernel(x, tkn_emb_table, W, b):
    Bb, S = x.shape
    V, E = tkn_emb_table.shape
    tb = tkn_emb_table.astype(jnp.bfloat16)
    Wb = W.astype(jnp.bfloat16)
    b2 = b.reshape(V, 1)
    idx_t = x.T.reshape(-1)
    chunks = (5,) * 10
    assert sum(chunks) == S
    offs, gathers, off = [], [], 0
    for ch in chunks:
        offs.append(off)
        gathers.append(
            _sc_gather(tb, lax.slice(idx_t, (off * Bb,), ((off + ch) * Bb,)))
            .reshape(ch, Bb, E)
        )
        off += ch
    out_t = None
    for off_k, Gk in zip(offs, gathers):
        out_t = _tc_proj_chunk(Wb, Gk, b2, S, off_k, out_t)
    return jnp.transpose(out_t, (2, 0, 1))


# speedup vs baseline: 1.1262x; 1.1262x over previous
import functools

import jax
import jax.numpy as jnp
from jax import lax
from jax.experimental import pallas as pl
from jax.experimental.pallas import tpu as pltpu
from jax.experimental.pallas import tpu_sc as plsc


def _sc_gather(table, flat_idx):
    info = plsc.get_sparse_core_info()
    NC, NS = info.num_cores, info.num_subcores
    NW = NC * NS
    N = flat_idx.shape[0]
    E = table.shape[1]
    b_per_w = N // NW
    CH = 80
    n_ch = b_per_w // CH
    assert b_per_w * NW == N and n_ch * CH == b_per_w

    mesh = plsc.VectorSubcoreMesh(core_axis_name="c", subcore_axis_name="s")

    @functools.partial(
        pl.kernel,
        out_type=jax.ShapeDtypeStruct((N, E), table.dtype),
        mesh=mesh,
        scratch_types=[
            pltpu.VMEM((b_per_w,), jnp.int32),
            pltpu.VMEM((2, CH, E), table.dtype),
            pltpu.SemaphoreType.DMA,
            pltpu.SemaphoreType.DMA,
        ],
        compiler_params=pltpu.CompilerParams(use_tc_tiling_on_sc=False),
    )
    def k(tab_hbm, idx_hbm, out_hbm, idx_v, rows_v, sem0, sem1):
        wid = lax.axis_index("s") * NC + lax.axis_index("c")
        base = wid * b_per_w
        pltpu.sync_copy(idx_hbm.at[pl.ds(base, b_per_w)], idx_v)
        sems = (sem0, sem1)

        def start(c, slot):
            pltpu.async_copy(
                tab_hbm.at[idx_v.at[pl.ds(c * CH, CH)]],
                rows_v.at[slot],
                sems[slot],
            )

        def finish(c, slot):
            pltpu.make_async_copy(
                tab_hbm.at[idx_v.at[pl.ds(c * CH, CH)]],
                rows_v.at[slot],
                sems[slot],
            ).wait()
            pltpu.sync_copy(
                rows_v.at[slot], out_hbm.at[pl.ds(base + c * CH, CH)]
            )

        start(0, 0)
        start(1, 1)

        def body(g, carry):
            for j in range(2):
                c = 2 * g + j
                finish(c, j)

                @pl.when(c + 2 < n_ch)
                def _():
                    start(c + 2, j)

            return carry

        lax.fori_loop(0, n_ch // 2, body, 0)

    return k(table, flat_idx)


def _proj_body(w_ref, g_ref, b_ref, out_ref):
    g = g_ref[0]
    acc = lax.dot_general(
        w_ref[...], g,
        dimension_numbers=(((1,), (1,)), ((), ())),
        preferred_element_type=jnp.float32,
    )
    out_ref[0] = acc + b_ref[...]


def _proj_body_alias(w_ref, g_ref, b_ref, alias_ref, out_ref):
    del alias_ref
    _proj_body(w_ref, g_ref, b_ref, out_ref)


def _tc_proj_chunk(Wb, Gk, b2, S, off, out_prev=None):
    S_CH, Bb, E = Gk.shape
    V = Wb.shape[0]
    in_specs = [
        pl.BlockSpec((V, E), lambda s: (0, 0)),
        pl.BlockSpec((1, Bb, E), lambda s: (s, 0, 0)),
        pl.BlockSpec((V, 1), lambda s: (0, 0)),
    ]
    args = [Wb, Gk, b2]
    body = _proj_body
    aliases = {}
    if out_prev is not None:
        in_specs.append(pl.BlockSpec(memory_space=pl.ANY))
        args.append(out_prev)
        body = _proj_body_alias
        aliases = {3: 0}
    return pl.pallas_call(
        body,
        grid=(S_CH,),
        in_specs=in_specs,
        out_specs=pl.BlockSpec((1, V, Bb), lambda s, off=off: (s + off, 0, 0)),
        out_shape=jax.ShapeDtypeStruct((S, V, Bb), jnp.float32),
        input_output_aliases=aliases,
    )(*args)


def kernel(x, tkn_emb_table, W, b):
    Bb, S = x.shape
    V, E = tkn_emb_table.shape
    tb = tkn_emb_table.astype(jnp.bfloat16)
    Wb = W.astype(jnp.bfloat16)
    b2 = b.reshape(V, 1)
    idx_t = x.T.reshape(-1)
    chunks = (10,) * 5
    assert sum(chunks) == S
    offs, gathers, off = [], [], 0
    for ch in chunks:
        offs.append(off)
        gathers.append(
            _sc_gather(tb, lax.slice(idx_t, (off * Bb,), ((off + ch) * Bb,)))
            .reshape(ch, Bb, E)
        )
        off += ch
    out_t = None
    for off_k, Gk in zip(offs, gathers):
        out_t = _tc_proj_chunk(Wb, Gk, b2, S, off_k, out_t)
    return jnp.transpose(out_t, (2, 0, 1))


# speedup vs baseline: 1.1800x; 1.0477x over previous
import functools

import jax
import jax.numpy as jnp
from jax import lax
from jax.experimental import pallas as pl
from jax.experimental.pallas import tpu as pltpu
from jax.experimental.pallas import tpu_sc as plsc


def _sc_gather(table, flat_idx):
    info = plsc.get_sparse_core_info()
    NC, NS = info.num_cores, info.num_subcores
    NW = NC * NS
    N = flat_idx.shape[0]
    E = table.shape[1]
    b_per_w = N // NW
    CH = 80
    n_ch = b_per_w // CH
    assert b_per_w * NW == N and n_ch * CH == b_per_w

    mesh = plsc.VectorSubcoreMesh(core_axis_name="c", subcore_axis_name="s")

    @functools.partial(
        pl.kernel,
        out_type=jax.ShapeDtypeStruct((N, E), table.dtype),
        mesh=mesh,
        scratch_types=[
            pltpu.VMEM((b_per_w,), jnp.int32),
            pltpu.VMEM((2, CH, E), table.dtype),
            pltpu.SemaphoreType.DMA,
            pltpu.SemaphoreType.DMA,
        ],
        compiler_params=pltpu.CompilerParams(use_tc_tiling_on_sc=False),
    )
    def k(tab_hbm, idx_hbm, out_hbm, idx_v, rows_v, sem0, sem1):
        wid = lax.axis_index("s") * NC + lax.axis_index("c")
        base = wid * b_per_w
        pltpu.sync_copy(idx_hbm.at[pl.ds(base, b_per_w)], idx_v)
        sems = (sem0, sem1)

        def start(c, slot):
            pltpu.async_copy(
                tab_hbm.at[idx_v.at[pl.ds(c * CH, CH)]],
                rows_v.at[slot],
                sems[slot],
            )

        def finish(c, slot):
            pltpu.make_async_copy(
                tab_hbm.at[idx_v.at[pl.ds(c * CH, CH)]],
                rows_v.at[slot],
                sems[slot],
            ).wait()
            pltpu.sync_copy(
                rows_v.at[slot], out_hbm.at[pl.ds(base + c * CH, CH)]
            )

        start(0, 0)
        start(1, 1)

        def body(g, carry):
            for j in range(2):
                c = 2 * g + j
                finish(c, j)

                @pl.when(c + 2 < n_ch)
                def _():
                    start(c + 2, j)

            return carry

        lax.fori_loop(0, n_ch // 2, body, 0)

    return k(table, flat_idx)


def _proj_body(w_ref, g_ref, b_ref, out_ref):
    for i in range(g_ref.shape[0]):
        g = g_ref[i]
        acc = lax.dot_general(
            w_ref[...], g,
            dimension_numbers=(((1,), (1,)), ((), ())),
            preferred_element_type=jnp.float32,
        )
        out_ref[i] = acc + b_ref[...]


def _proj_body_alias(w_ref, g_ref, b_ref, alias_ref, out_ref):
    del alias_ref
    _proj_body(w_ref, g_ref, b_ref, out_ref)


def _tc_proj_chunk(Wb, Gk, b2, S, off, out_prev=None, sb=1):
    S_CH, Bb, E = Gk.shape
    V = Wb.shape[0]
    in_specs = [
        pl.BlockSpec((V, E), lambda s: (0, 0)),
        pl.BlockSpec((sb, Bb, E), lambda s: (s, 0, 0)),
        pl.BlockSpec((V, 1), lambda s: (0, 0)),
    ]
    args = [Wb, Gk, b2]
    body = _proj_body
    aliases = {}
    if out_prev is not None:
        in_specs.append(pl.BlockSpec(memory_space=pl.ANY))
        args.append(out_prev)
        body = _proj_body_alias
        aliases = {3: 0}
    noff = off // sb
    return pl.pallas_call(
        body,
        grid=(S_CH // sb,),
        in_specs=in_specs,
        out_specs=pl.BlockSpec((sb, V, Bb), lambda s, noff=noff: (s + noff, 0, 0)),
        out_shape=jax.ShapeDtypeStruct((S, V, Bb), jnp.float32),
        input_output_aliases=aliases,
    )(*args)


def kernel(x, tkn_emb_table, W, b):
    Bb, S = x.shape
    V, E = tkn_emb_table.shape
    tb = tkn_emb_table.astype(jnp.bfloat16)
    Wb = W.astype(jnp.bfloat16)
    b2 = b.reshape(V, 1)
    idx_t = x.T.reshape(-1)
    chunks = (10,) * 5
    assert sum(chunks) == S
    offs, gathers, off = [], [], 0
    for ch in chunks:
        offs.append(off)
        gathers.append(
            _sc_gather(tb, lax.slice(idx_t, (off * Bb,), ((off + ch) * Bb,)))
            .reshape(ch, Bb, E)
        )
        off += ch
    out_t = None
    for off_k, Gk in zip(offs, gathers):
        out_t = _tc_proj_chunk(Wb, Gk, b2, S, off_k, out_t, sb=2)
    return jnp.transpose(out_t, (2, 0, 1))
